# Initial kernel scaffold; baseline (speedup 1.0000x reference)
import jax
import jax.numpy as jnp
from jax.experimental import pallas as pl


def kernel(x_user, x_repo, edge_index_fwd, edge_index_rev, edge_label_index, Wl1f, bl1f, Wr1f, Wl1r, bl1r, Wr1r, Wl2f, bl2f, Wr2f, Wl2r, bl2r, Wr2r, Wu, bu, Wp, bp):
    raise NotImplementedError("write your pallas kernel here")


# speedup vs baseline: 4.7106x; 4.7106x over previous
import functools

import jax
import jax.numpy as jnp
from jax import lax
from jax.experimental import pallas as pl
from jax.experimental.pallas import tpu as pltpu
from jax.experimental.pallas import tpu_sc as plsc

NC = 2
NS = 16
NW = NC * NS
L = 16
C = 128

_HIGH = lax.Precision.HIGHEST


def _part8(N):
    r0 = (-(-N // NS) + 7) // 8 * 8
    rl = N - (NS - 1) * r0
    assert rl > 0 and rl % 8 == 0
    return r0, rl


def _mesh():
    return plsc.VectorSubcoreMesh(
        core_axis_name="c", subcore_axis_name="s", num_cores=NC, num_subcores=NS)


def _make_sage_pass(NU, NR, D, E, with_cnt):
    assert E % C == 0
    n_chunk = E // C
    k_max = (n_chunk + NW - 1) // NW

    r0R, rlR = _part8(NR)
    r0U, rlU = _part8(NU)
    NRp = -(-NR // 128) * 128
    NUp = -(-NU // 128) * 128

    out_type = [
        jax.ShapeDtypeStruct((NC * NR, D), jnp.float32),
        jax.ShapeDtypeStruct((NC * NU, D), jnp.float32),
    ]
    scratch = [
        pltpu.VMEM_SHARED((NR, D), jnp.float32),
        pltpu.VMEM_SHARED((NU, D), jnp.float32),
        pltpu.VMEM((C,), jnp.int32),
        pltpu.VMEM((C,), jnp.int32),
        pltpu.VMEM((C, D), jnp.float32),
        pltpu.SemaphoreType.DMA,
    ]
    if with_cnt:
        out_type += [
            jax.ShapeDtypeStruct((NRp,), jnp.float32),
            jax.ShapeDtypeStruct((NRp,), jnp.float32),
            jax.ShapeDtypeStruct((NUp,), jnp.float32),
            jax.ShapeDtypeStruct((NUp,), jnp.float32),
        ]
        scratch += [
            pltpu.VMEM_SHARED((NRp,), jnp.float32),
            pltpu.VMEM_SHARED((NUp,), jnp.float32),
            pltpu.VMEM((C,), jnp.float32),
        ]

    def body(*refs):
        if with_cnt:
            (xa, xb, srcf, dstf, srcr, dstr, zrow, zcR, zcU, ones_h,
             outR, outU, ocR0, ocR1, ocU0, ocU1,
             accR, accU, sidx, didx, rows, gsem, cacR, cacU, ones) = refs
        else:
            (xa, xb, srcf, dstf, srcr, dstr, zrow,
             outR, outU,
             accR, accU, sidx, didx, rows, gsem) = refs
        c = lax.axis_index("c")
        s = lax.axis_index("s")
        wid = s * NC + c

        def _rowcopy(get_src, get_dst, r0, rl):
            @pl.when(s < NS - 1)
            def _():
                pltpu.sync_copy(get_src(s * r0, r0), get_dst(s * r0, r0))

            @pl.when(s == NS - 1)
            def _():
                pltpu.sync_copy(get_src((NS - 1) * r0, rl),
                                get_dst((NS - 1) * r0, rl))

        _rowcopy(lambda o, n: zrow.at[pl.ds(0, n)],
                 lambda o, n: accR.at[pl.ds(o, n)], r0R, rlR)
        _rowcopy(lambda o, n: zrow.at[pl.ds(0, n)],
                 lambda o, n: accU.at[pl.ds(o, n)], r0U, rlU)
        if with_cnt:
            @pl.when(s == 0)
            def _():
                pltpu.sync_copy(zcR, cacR)

            @pl.when(s == 1)
            def _():
                pltpu.sync_copy(zcU, cacU)

            pltpu.sync_copy(ones_h, ones)
        plsc.subcore_barrier()

        def do_dir(src_h, dst_h, x_h, acc, cacc):
            def step(k, carry):
                g = wid + k * NW

                @pl.when(g < n_chunk)
                def _():
                    base = g * C
                    pltpu.sync_copy(src_h.at[pl.ds(base, C)], sidx)
                    pltpu.sync_copy(dst_h.at[pl.ds(base, C)], didx)
                    pltpu.async_copy(x_h.at[sidx], rows, gsem).wait()
                    pltpu.sync_copy(rows, acc.at[didx], add=True)
                    if cacc is not None:
                        pltpu.sync_copy(ones, cacc.at[didx], add=True)

                return carry

            lax.fori_loop(0, k_max, step, 0)

        do_dir(srcf, dstf, xa, accR, cacR if with_cnt else None)
        do_dir(srcr, dstr, xb, accU, cacU if with_cnt else None)
        plsc.subcore_barrier()

        _rowcopy(lambda o, n: accR.at[pl.ds(o, n)],
                 lambda o, n: outR.at[pl.ds(c * NR + o, n)], r0R, rlR)
        _rowcopy(lambda o, n: accU.at[pl.ds(o, n)],
                 lambda o, n: outU.at[pl.ds(c * NU + o, n)], r0U, rlU)
        if with_cnt:
            @pl.when(jnp.logical_and(s == 0, c == 0))
            def _():
                pltpu.sync_copy(cacR, ocR0)

            @pl.when(jnp.logical_and(s == 0, c == 1))
            def _():
                pltpu.sync_copy(cacR, ocR1)

            @pl.when(jnp.logical_and(s == 1, c == 0))
            def _():
                pltpu.sync_copy(cacU, ocU0)

            @pl.when(jnp.logical_and(s == 1, c == 1))
            def _():
                pltpu.sync_copy(cacU, ocU1)

    return pl.kernel(body, out_type=out_type, mesh=_mesh(), scratch_types=scratch)


def _make_decode(NG, EL, NRR):
    n_full = EL // C
    tail = EL - n_full * C
    k_max = (n_full + NW - 1) // NW
    assert tail % L == 0 and (n_full * C) % 8 == 0

    scratch = [
        pltpu.VMEM((C,), jnp.int32),
        pltpu.VMEM((C,), jnp.int32),
        pltpu.VMEM((C,), jnp.int32),
        pltpu.VMEM((C,), jnp.float32),
        pltpu.SemaphoreType.DMA,
    ]

    def body(g_h, row_h, col_h, out_h, ridx, cidx, fidx, vals, sem):
        c = lax.axis_index("c")
        s = lax.axis_index("s")
        wid = s * NC + c

        def do_chunk(base, n):
            pltpu.sync_copy(row_h.at[pl.ds(base, n)], ridx.at[pl.ds(0, n)])
            pltpu.sync_copy(col_h.at[pl.ds(base, n)], cidx.at[pl.ds(0, n)])
            for j in range(n // L):
                sl = pl.ds(j * L, L)
                fidx[sl] = ridx[sl] * NRR + cidx[sl]
            pltpu.async_copy(g_h.at[fidx], vals, sem).wait()
            pltpu.sync_copy(vals.at[pl.ds(0, n)], out_h.at[pl.ds(base, n)])

        def step(k, carry):
            g = wid + k * NW

            @pl.when(g < n_full)
            def _():
                do_chunk(g * C, C)

            return carry

        lax.fori_loop(0, k_max, step, 0)
        if tail:
            @pl.when(wid == NW - 1)
            def _():
                base = n_full * C
                pltpu.sync_copy(row_h.at[pl.ds(base, tail)], ridx.at[pl.ds(0, tail)])
                pltpu.sync_copy(col_h.at[pl.ds(base, tail)], cidx.at[pl.ds(0, tail)])
                for j in range(tail // L):
                    sl = pl.ds(j * L, L)
                    fidx[sl] = ridx[sl] * NRR + cidx[sl]
                pltpu.async_copy(g_h.at[fidx.at[pl.ds(0, tail)]], vals.at[pl.ds(0, tail)], sem).wait()
                pltpu.sync_copy(vals.at[pl.ds(0, tail)], out_h.at[pl.ds(base, tail)])

    return pl.kernel(
        body,
        out_type=jax.ShapeDtypeStruct((EL,), jnp.float32),
        mesh=_mesh(),
        scratch_types=scratch,
    )


def _side_affine(agg_ref, cnt0_ref, cnt1_ref, x_ref, Wl_ref, bl_ref, Wr_ref):
    N = agg_ref.shape[1]
    agg = agg_ref[0] + agg_ref[1]
    cnt = jnp.maximum(cnt0_ref[...][:N] + cnt1_ref[...][:N], 1.0)
    agg = agg / cnt
    return (lax.dot_general(agg, Wl_ref[...], (((1,), (1,)), ((), ())),
                            precision=_HIGH, preferred_element_type=jnp.float32)
            + bl_ref[...]
            + lax.dot_general(x_ref[...], Wr_ref[...], (((1,), (1,)), ((), ())),
                              precision=_HIGH, preferred_element_type=jnp.float32))


def _tc_side(N, D, relu):
    def body(agg, cnt0, cnt1, x, Wl, bl, Wr, o):
        h = _side_affine(agg, cnt0, cnt1, x, Wl, bl, Wr)
        o[...] = jnp.maximum(h, 0.0) if relu else h

    return pl.pallas_call(
        body, out_shape=jax.ShapeDtypeStruct((N, D), jnp.float32))


def _tc_side_head(N, D):
    def body(agg, cnt0, cnt1, x, Wl, bl, Wr, Wh, bh, o):
        h = _side_affine(agg, cnt0, cnt1, x, Wl, bl, Wr)
        o[...] = (lax.dot_general(h, Wh[...], (((1,), (1,)), ((), ())),
                                  precision=_HIGH, preferred_element_type=jnp.float32)
                  + bh[...])

    return pl.pallas_call(
        body, out_shape=jax.ShapeDtypeStruct((N, D), jnp.float32))


def _tc_gram(NU, NR, D, blk=1000):
    assert NU % blk == 0

    def body(u_ref, p_ref, o_ref):
        o_ref[...] = lax.dot_general(u_ref[...], p_ref[...], (((1,), (1,)), ((), ())),
                                     precision=_HIGH, preferred_element_type=jnp.float32)

    return pl.pallas_call(
        body,
        grid=(NU // blk,),
        in_specs=[pl.BlockSpec((blk, D), lambda i: (i, 0)),
                  pl.BlockSpec((NR, D), lambda i: (0, 0))],
        out_specs=pl.BlockSpec((blk, NR), lambda i: (i, 0)),
        out_shape=jax.ShapeDtypeStruct((NU, NR), jnp.float32),
    )


def kernel(x_user, x_repo, edge_index_fwd, edge_index_rev, edge_label_index,
           Wl1f, bl1f, Wr1f, Wl1r, bl1r, Wr1r,
           Wl2f, bl2f, Wr2f, Wl2r, bl2r, Wr2r,
           Wu, bu, Wp, bp):
    NU, D = x_user.shape
    NR = x_repo.shape[0]
    E = edge_index_fwd.shape[1]
    EL = edge_label_index.shape[1]

    srcf = edge_index_fwd[0].astype(jnp.int32)
    dstf = edge_index_fwd[1].astype(jnp.int32)
    srcr = edge_index_rev[0].astype(jnp.int32)
    dstr = edge_index_rev[1].astype(jnp.int32)
    row = edge_label_index[0].astype(jnp.int32)
    col = edge_label_index[1].astype(jnp.int32)

    NRp = -(-NR // 128) * 128
    NUp = -(-NU // 128) * 128
    zrow = jnp.zeros((max(_part8(NU)[0], _part8(NR)[0]), D), jnp.float32)
    zcR = jnp.zeros((NRp,), jnp.float32)
    zcU = jnp.zeros((NUp,), jnp.float32)
    ones_h = jnp.ones((C,), jnp.float32)

    sage1 = _make_sage_pass(NU, NR, D, E, with_cnt=True)
    aggR1, aggU1, cR0, cR1, cU0, cU1 = sage1(
        x_user, x_repo, srcf, dstf, srcr, dstr, zrow, zcR, zcU, ones_h)
    aggR1 = aggR1.reshape(NC, NR, D)
    aggU1 = aggU1.reshape(NC, NU, D)
    cR0, cR1 = cR0.reshape(NRp, 1), cR1.reshape(NRp, 1)
    cU0, cU1 = cU0.reshape(NUp, 1), cU1.reshape(NUp, 1)

    repo1 = _tc_side(NR, D, relu=True)(
        aggR1, cR0, cR1, x_repo, Wl1f, bl1f.reshape(1, D), Wr1f)
    user1 = _tc_side(NU, D, relu=True)(
        aggU1, cU0, cU1, x_user, Wl1r, bl1r.reshape(1, D), Wr1r)

    sage2 = _make_sage_pass(NU, NR, D, E, with_cnt=False)
    aggR2, aggU2 = sage2(user1, repo1, srcf, dstf, srcr, dstr, zrow)
    aggR2 = aggR2.reshape(NC, NR, D)
    aggU2 = aggU2.reshape(NC, NU, D)

    P = _tc_side_head(NR, D)(
        aggR2, cR0, cR1, repo1, Wl2f, bl2f.reshape(1, D), Wr2f,
        Wp, bp.reshape(1, D))
    U = _tc_side_head(NU, D)(
        aggU2, cU0, cU1, user1, Wl2r, bl2r.reshape(1, D), Wr2r,
        Wu, bu.reshape(1, D))

    G = _tc_gram(NU, NR, D)(U, P)

    out = _make_decode(NU * NR, EL, NR)(G.reshape(-1), row, col)
    return out


# speedup vs baseline: 6.5383x; 1.3880x over previous
import functools

import jax
import jax.numpy as jnp
from jax import lax
from jax.experimental import pallas as pl
from jax.experimental.pallas import tpu as pltpu
from jax.experimental.pallas import tpu_sc as plsc

NC = 2
NS = 16
NW = NC * NS
L = 16
C = 128
NB = 2

_HIGH = lax.Precision.HIGHEST


def _part8(N):
    r0 = (-(-N // NS) + 7) // 8 * 8
    rl = N - (NS - 1) * r0
    assert rl > 0 and rl % 8 == 0
    return r0, rl


def _mesh():
    return plsc.VectorSubcoreMesh(
        core_axis_name="c", subcore_axis_name="s", num_cores=NC, num_subcores=NS)


def _make_sage_pass(NU, NR, D, E, with_cnt):
    assert E % C == 0
    n_chunk = E // C
    k_max = (n_chunk + NW - 1) // NW

    r0R, rlR = _part8(NR)
    r0U, rlU = _part8(NU)
    NRp = -(-NR // 128) * 128
    NUp = -(-NU // 128) * 128

    out_type = [
        jax.ShapeDtypeStruct((NC * NR, D), jnp.float32),
        jax.ShapeDtypeStruct((NC * NU, D), jnp.float32),
    ]
    scratch = [
        pltpu.VMEM_SHARED((NR, D), jnp.float32),
        pltpu.VMEM_SHARED((NU, D), jnp.float32),
        pltpu.VMEM((NB, C), jnp.int32),
        pltpu.VMEM((NB, C), jnp.int32),
        pltpu.VMEM((NB, C, D), jnp.float32),
        pltpu.SemaphoreType.DMA((NB,)),
        pltpu.SemaphoreType.DMA((NB,)),
        pltpu.SemaphoreType.DMA((NB,)),
    ]
    if with_cnt:
        out_type += [
            jax.ShapeDtypeStruct((NRp,), jnp.float32),
            jax.ShapeDtypeStruct((NRp,), jnp.float32),
            jax.ShapeDtypeStruct((NUp,), jnp.float32),
            jax.ShapeDtypeStruct((NUp,), jnp.float32),
        ]
        scratch += [
            pltpu.VMEM_SHARED((NRp,), jnp.float32),
            pltpu.VMEM_SHARED((NUp,), jnp.float32),
            pltpu.VMEM((C,), jnp.float32),
            pltpu.SemaphoreType.DMA((NB,)),
        ]

    def body(*refs):
        csem = cacR = cacU = ones = None
        if with_cnt:
            (xa, xb, srcf, dstf, srcr, dstr, zrow, zcR, zcU, ones_h,
             outR, outU, ocR0, ocR1, ocU0, ocU1,
             accR, accU, sidx, didx, rows, isem, gsem, ssem,
             cacR, cacU, ones, csem) = refs
        else:
            (xa, xb, srcf, dstf, srcr, dstr, zrow,
             outR, outU,
             accR, accU, sidx, didx, rows, isem, gsem, ssem) = refs
        c = lax.axis_index("c")
        s = lax.axis_index("s")
        wid = s * NC + c

        def _rowcopy(get_src, get_dst, r0, rl):
            @pl.when(s < NS - 1)
            def _():
                pltpu.sync_copy(get_src(s * r0, r0), get_dst(s * r0, r0))

            @pl.when(s == NS - 1)
            def _():
                pltpu.sync_copy(get_src((NS - 1) * r0, rl),
                                get_dst((NS - 1) * r0, rl))

        _rowcopy(lambda o, n: zrow.at[pl.ds(0, n)],
                 lambda o, n: accR.at[pl.ds(o, n)], r0R, rlR)
        _rowcopy(lambda o, n: zrow.at[pl.ds(0, n)],
                 lambda o, n: accU.at[pl.ds(o, n)], r0U, rlU)
        if with_cnt:
            @pl.when(s == 0)
            def _():
                pltpu.sync_copy(zcR, cacR)

            @pl.when(s == 1)
            def _():
                pltpu.sync_copy(zcU, cacU)

            pltpu.sync_copy(ones_h, ones)
        plsc.subcore_barrier()

        def do_dir(src_h, dst_h, x_h, acc, cacc):
            n_outer = (k_max + NB - 1) // NB

            def drain_scatter(b):
                pltpu.make_async_copy(rows.at[b], acc.at[didx.at[b]], ssem.at[b]).wait()
                if cacc is not None:
                    pltpu.make_async_copy(ones, cacc.at[didx.at[b]], csem.at[b]).wait()

            def step(k, carry):
                for b in range(NB):
                    j = k * NB + b
                    g = wid + j * NW
                    gprev = g - NB * NW

                    @pl.when(jnp.logical_and(j >= NB, gprev < n_chunk))
                    def _():
                        drain_scatter(b)

                    @pl.when(g < n_chunk)
                    def _():
                        base = g * C
                        pltpu.async_copy(src_h.at[pl.ds(base, C)], sidx.at[b], isem.at[b])
                        pltpu.async_copy(dst_h.at[pl.ds(base, C)], didx.at[b], isem.at[b])

                for b in range(NB):
                    g = wid + (k * NB + b) * NW

                    @pl.when(g < n_chunk)
                    def _():
                        pltpu.make_async_copy(src_h.at[pl.ds(0, C)], sidx.at[b], isem.at[b]).wait()
                        pltpu.make_async_copy(dst_h.at[pl.ds(0, C)], didx.at[b], isem.at[b]).wait()
                        pltpu.async_copy(x_h.at[sidx.at[b]], rows.at[b], gsem.at[b])

                for b in range(NB):
                    g = wid + (k * NB + b) * NW

                    @pl.when(g < n_chunk)
                    def _():
                        pltpu.make_async_copy(x_h.at[sidx.at[b]], rows.at[b], gsem.at[b]).wait()
                        pltpu.async_copy(rows.at[b], acc.at[didx.at[b]], ssem.at[b], add=True)
                        if cacc is not None:
                            pltpu.async_copy(ones, cacc.at[didx.at[b]], csem.at[b], add=True)

                return carry

            lax.fori_loop(0, n_outer, step, 0)
            for b in range(NB):
                g = wid + ((n_outer - 1) * NB + b) * NW

                @pl.when(g < n_chunk)
                def _():
                    drain_scatter(b)

        do_dir(srcf, dstf, xa, accR, cacR if with_cnt else None)
        do_dir(srcr, dstr, xb, accU, cacU if with_cnt else None)
        plsc.subcore_barrier()

        _rowcopy(lambda o, n: accR.at[pl.ds(o, n)],
                 lambda o, n: outR.at[pl.ds(c * NR + o, n)], r0R, rlR)
        _rowcopy(lambda o, n: accU.at[pl.ds(o, n)],
                 lambda o, n: outU.at[pl.ds(c * NU + o, n)], r0U, rlU)
        if with_cnt:
            @pl.when(jnp.logical_and(s == 0, c == 0))
            def _():
                pltpu.sync_copy(cacR, ocR0)

            @pl.when(jnp.logical_and(s == 0, c == 1))
            def _():
                pltpu.sync_copy(cacR, ocR1)

            @pl.when(jnp.logical_and(s == 1, c == 0))
            def _():
                pltpu.sync_copy(cacU, ocU0)

            @pl.when(jnp.logical_and(s == 1, c == 1))
            def _():
                pltpu.sync_copy(cacU, ocU1)

    return pl.kernel(body, out_type=out_type, mesh=_mesh(), scratch_types=scratch)


def _make_decode(NG, EL, NRR):
    n_full = EL // C
    tail = EL - n_full * C
    k_max = (n_full + NW - 1) // NW
    assert tail % L == 0 and (n_full * C) % 8 == 0

    scratch = [
        pltpu.VMEM((C,), jnp.int32),
        pltpu.VMEM((C,), jnp.int32),
        pltpu.VMEM((C,), jnp.int32),
        pltpu.VMEM((C,), jnp.float32),
        pltpu.SemaphoreType.DMA,
    ]

    def body(g_h, row_h, col_h, out_h, ridx, cidx, fidx, vals, sem):
        c = lax.axis_index("c")
        s = lax.axis_index("s")
        wid = s * NC + c

        def do_chunk(base, n):
            pltpu.sync_copy(row_h.at[pl.ds(base, n)], ridx.at[pl.ds(0, n)])
            pltpu.sync_copy(col_h.at[pl.ds(base, n)], cidx.at[pl.ds(0, n)])
            for j in range(n // L):
                sl = pl.ds(j * L, L)
                fidx[sl] = ridx[sl] * NRR + cidx[sl]
            pltpu.async_copy(g_h.at[fidx], vals, sem).wait()
            pltpu.sync_copy(vals.at[pl.ds(0, n)], out_h.at[pl.ds(base, n)])

        def step(k, carry):
            g = wid + k * NW

            @pl.when(g < n_full)
            def _():
                do_chunk(g * C, C)

            return carry

        lax.fori_loop(0, k_max, step, 0)
        if tail:
            @pl.when(wid == NW - 1)
            def _():
                base = n_full * C
                pltpu.sync_copy(row_h.at[pl.ds(base, tail)], ridx.at[pl.ds(0, tail)])
                pltpu.sync_copy(col_h.at[pl.ds(base, tail)], cidx.at[pl.ds(0, tail)])
                for j in range(tail // L):
                    sl = pl.ds(j * L, L)
                    fidx[sl] = ridx[sl] * NRR + cidx[sl]
                pltpu.async_copy(g_h.at[fidx.at[pl.ds(0, tail)]], vals.at[pl.ds(0, tail)], sem).wait()
                pltpu.sync_copy(vals.at[pl.ds(0, tail)], out_h.at[pl.ds(base, tail)])

    return pl.kernel(
        body,
        out_type=jax.ShapeDtypeStruct((EL,), jnp.float32),
        mesh=_mesh(),
        scratch_types=scratch,
    )


def _side_affine(agg_ref, cnt0_ref, cnt1_ref, x_ref, Wl_ref, bl_ref, Wr_ref):
    N = agg_ref.shape[1]
    agg = agg_ref[0] + agg_ref[1]
    cnt = jnp.maximum(cnt0_ref[...][:N] + cnt1_ref[...][:N], 1.0)
    agg = agg / cnt
    return (lax.dot_general(agg, Wl_ref[...], (((1,), (1,)), ((), ())),
                            precision=_HIGH, preferred_element_type=jnp.float32)
            + bl_ref[...]
            + lax.dot_general(x_ref[...], Wr_ref[...], (((1,), (1,)), ((), ())),
                              precision=_HIGH, preferred_element_type=jnp.float32))


def _tc_side(N, D, relu):
    def body(agg, cnt0, cnt1, x, Wl, bl, Wr, o):
        h = _side_affine(agg, cnt0, cnt1, x, Wl, bl, Wr)
        o[...] = jnp.maximum(h, 0.0) if relu else h

    return pl.pallas_call(
        body, out_shape=jax.ShapeDtypeStruct((N, D), jnp.float32))


def _tc_side_head(N, D):
    def body(agg, cnt0, cnt1, x, Wl, bl, Wr, Wh, bh, o):
        h = _side_affine(agg, cnt0, cnt1, x, Wl, bl, Wr)
        o[...] = (lax.dot_general(h, Wh[...], (((1,), (1,)), ((), ())),
                                  precision=_HIGH, preferred_element_type=jnp.float32)
                  + bh[...])

    return pl.pallas_call(
        body, out_shape=jax.ShapeDtypeStruct((N, D), jnp.float32))


def _tc_gram(NU, NR, D, blk=1000):
    assert NU % blk == 0

    def body(u_ref, p_ref, o_ref):
        o_ref[...] = lax.dot_general(u_ref[...], p_ref[...], (((1,), (1,)), ((), ())),
                                     precision=_HIGH, preferred_element_type=jnp.float32)

    return pl.pallas_call(
        body,
        grid=(NU // blk,),
        in_specs=[pl.BlockSpec((blk, D), lambda i: (i, 0)),
                  pl.BlockSpec((NR, D), lambda i: (0, 0))],
        out_specs=pl.BlockSpec((blk, NR), lambda i: (i, 0)),
        out_shape=jax.ShapeDtypeStruct((NU, NR), jnp.float32),
    )


def kernel(x_user, x_repo, edge_index_fwd, edge_index_rev, edge_label_index,
           Wl1f, bl1f, Wr1f, Wl1r, bl1r, Wr1r,
           Wl2f, bl2f, Wr2f, Wl2r, bl2r, Wr2r,
           Wu, bu, Wp, bp):
    NU, D = x_user.shape
    NR = x_repo.shape[0]
    E = edge_index_fwd.shape[1]
    EL = edge_label_index.shape[1]

    srcf = edge_index_fwd[0].astype(jnp.int32)
    dstf = edge_index_fwd[1].astype(jnp.int32)
    srcr = edge_index_rev[0].astype(jnp.int32)
    dstr = edge_index_rev[1].astype(jnp.int32)
    row = edge_label_index[0].astype(jnp.int32)
    col = edge_label_index[1].astype(jnp.int32)

    NRp = -(-NR // 128) * 128
    NUp = -(-NU // 128) * 128
    zrow = jnp.zeros((max(_part8(NU)[0], _part8(NR)[0]), D), jnp.float32)
    zcR = jnp.zeros((NRp,), jnp.float32)
    zcU = jnp.zeros((NUp,), jnp.float32)
    ones_h = jnp.ones((C,), jnp.float32)

    sage1 = _make_sage_pass(NU, NR, D, E, with_cnt=True)
    aggR1, aggU1, cR0, cR1, cU0, cU1 = sage1(
        x_user, x_repo, srcf, dstf, srcr, dstr, zrow, zcR, zcU, ones_h)
    aggR1 = aggR1.reshape(NC, NR, D)
    aggU1 = aggU1.reshape(NC, NU, D)
    cR0, cR1 = cR0.reshape(NRp, 1), cR1.reshape(NRp, 1)
    cU0, cU1 = cU0.reshape(NUp, 1), cU1.reshape(NUp, 1)

    repo1 = _tc_side(NR, D, relu=True)(
        aggR1, cR0, cR1, x_repo, Wl1f, bl1f.reshape(1, D), Wr1f)
    user1 = _tc_side(NU, D, relu=True)(
        aggU1, cU0, cU1, x_user, Wl1r, bl1r.reshape(1, D), Wr1r)

    sage2 = _make_sage_pass(NU, NR, D, E, with_cnt=False)
    aggR2, aggU2 = sage2(user1, repo1, srcf, dstf, srcr, dstr, zrow)
    aggR2 = aggR2.reshape(NC, NR, D)
    aggU2 = aggU2.reshape(NC, NU, D)

    P = _tc_side_head(NR, D)(
        aggR2, cR0, cR1, repo1, Wl2f, bl2f.reshape(1, D), Wr2f,
        Wp, bp.reshape(1, D))
    U = _tc_side_head(NU, D)(
        aggU2, cU0, cU1, user1, Wl2r, bl2r.reshape(1, D), Wr2r,
        Wu, bu.reshape(1, D))

    G = _tc_gram(NU, NR, D)(U, P)

    out = _make_decode(NU * NR, EL, NR)(G.reshape(-1), row, col)
    return out


# speedup vs baseline: 7.0641x; 1.0804x over previous
import functools

import jax
import jax.numpy as jnp
from jax import lax
from jax.experimental import pallas as pl
from jax.experimental.pallas import tpu as pltpu
from jax.experimental.pallas import tpu_sc as plsc

NC = 2
NS = 16
NW = NC * NS
L = 16
C = 128
NB = 3

_HIGH = lax.Precision.HIGHEST


def _part8(N):
    r0 = (-(-N // NS) + 7) // 8 * 8
    rl = N - (NS - 1) * r0
    assert rl > 0 and rl % 8 == 0
    return r0, rl


def _mesh():
    return plsc.VectorSubcoreMesh(
        core_axis_name="c", subcore_axis_name="s", num_cores=NC, num_subcores=NS)


def _make_sage_pass(NU, NR, D, E, with_cnt):
    assert E % C == 0
    n_chunk = E // C
    k_max = (n_chunk + NW - 1) // NW

    r0R, rlR = _part8(NR)
    r0U, rlU = _part8(NU)
    NRp = -(-NR // 128) * 128
    NUp = -(-NU // 128) * 128

    out_type = [
        jax.ShapeDtypeStruct((NC * NR, D), jnp.float32),
        jax.ShapeDtypeStruct((NC * NU, D), jnp.float32),
    ]
    scratch = [
        pltpu.VMEM_SHARED((NR, D), jnp.float32),
        pltpu.VMEM_SHARED((NU, D), jnp.float32),
        pltpu.VMEM((NB, C), jnp.int32),
        pltpu.VMEM((NB, C), jnp.int32),
        pltpu.VMEM((NB, C, D), jnp.float32),
        pltpu.SemaphoreType.DMA((NB,)),
        pltpu.SemaphoreType.DMA((NB,)),
        pltpu.SemaphoreType.DMA((NB,)),
    ]
    if with_cnt:
        out_type += [
            jax.ShapeDtypeStruct((NRp,), jnp.float32),
            jax.ShapeDtypeStruct((NRp,), jnp.float32),
            jax.ShapeDtypeStruct((NUp,), jnp.float32),
            jax.ShapeDtypeStruct((NUp,), jnp.float32),
        ]
        scratch += [
            pltpu.VMEM_SHARED((NRp,), jnp.float32),
            pltpu.VMEM_SHARED((NUp,), jnp.float32),
            pltpu.VMEM((C,), jnp.float32),
            pltpu.SemaphoreType.DMA((NB,)),
        ]

    def body(*refs):
        csem = cacR = cacU = ones = None
        if with_cnt:
            (xa, xb, srcf, dstf, srcr, dstr, zrow, zcR, zcU, ones_h,
             outR, outU, ocR0, ocR1, ocU0, ocU1,
             accR, accU, sidx, didx, rows, isem, gsem, ssem,
             cacR, cacU, ones, csem) = refs
        else:
            (xa, xb, srcf, dstf, srcr, dstr, zrow,
             outR, outU,
             accR, accU, sidx, didx, rows, isem, gsem, ssem) = refs
        c = lax.axis_index("c")
        s = lax.axis_index("s")
        wid = s * NC + c

        def _rowcopy(get_src, get_dst, r0, rl):
            @pl.when(s < NS - 1)
            def _():
                pltpu.sync_copy(get_src(s * r0, r0), get_dst(s * r0, r0))

            @pl.when(s == NS - 1)
            def _():
                pltpu.sync_copy(get_src((NS - 1) * r0, rl),
                                get_dst((NS - 1) * r0, rl))

        _rowcopy(lambda o, n: zrow.at[pl.ds(0, n)],
                 lambda o, n: accR.at[pl.ds(o, n)], r0R, rlR)
        _rowcopy(lambda o, n: zrow.at[pl.ds(0, n)],
                 lambda o, n: accU.at[pl.ds(o, n)], r0U, rlU)
        if with_cnt:
            @pl.when(s == 0)
            def _():
                pltpu.sync_copy(zcR, cacR)

            @pl.when(s == 1)
            def _():
                pltpu.sync_copy(zcU, cacU)

            pltpu.sync_copy(ones_h, ones)
        plsc.subcore_barrier()

        def do_dir(src_h, dst_h, x_h, acc, cacc):
            n_outer = (k_max + NB - 1) // NB

            def drain_scatter(b):
                pltpu.make_async_copy(rows.at[b], acc.at[didx.at[b]], ssem.at[b]).wait()
                if cacc is not None:
                    pltpu.make_async_copy(ones, cacc.at[didx.at[b]], csem.at[b]).wait()

            def step(k, carry):
                for b in range(NB):
                    j = k * NB + b
                    g = wid + j * NW
                    gprev = g - NB * NW

                    @pl.when(jnp.logical_and(j >= NB, gprev < n_chunk))
                    def _():
                        drain_scatter(b)

                    @pl.when(g < n_chunk)
                    def _():
                        base = g * C
                        pltpu.async_copy(src_h.at[pl.ds(base, C)], sidx.at[b], isem.at[b])
                        pltpu.async_copy(dst_h.at[pl.ds(base, C)], didx.at[b], isem.at[b])

                for b in range(NB):
                    g = wid + (k * NB + b) * NW

                    @pl.when(g < n_chunk)
                    def _():
                        pltpu.make_async_copy(src_h.at[pl.ds(0, C)], sidx.at[b], isem.at[b]).wait()
                        pltpu.make_async_copy(dst_h.at[pl.ds(0, C)], didx.at[b], isem.at[b]).wait()
                        pltpu.async_copy(x_h.at[sidx.at[b]], rows.at[b], gsem.at[b])

                for b in range(NB):
                    g = wid + (k * NB + b) * NW

                    @pl.when(g < n_chunk)
                    def _():
                        pltpu.make_async_copy(x_h.at[sidx.at[b]], rows.at[b], gsem.at[b]).wait()
                        pltpu.async_copy(rows.at[b], acc.at[didx.at[b]], ssem.at[b], add=True)
                        if cacc is not None:
                            pltpu.async_copy(ones, cacc.at[didx.at[b]], csem.at[b], add=True)

                return carry

            lax.fori_loop(0, n_outer, step, 0)
            for b in range(NB):
                g = wid + ((n_outer - 1) * NB + b) * NW

                @pl.when(g < n_chunk)
                def _():
                    drain_scatter(b)

        do_dir(srcf, dstf, xa, accR, cacR if with_cnt else None)
        do_dir(srcr, dstr, xb, accU, cacU if with_cnt else None)
        plsc.subcore_barrier()

        _rowcopy(lambda o, n: accR.at[pl.ds(o, n)],
                 lambda o, n: outR.at[pl.ds(c * NR + o, n)], r0R, rlR)
        _rowcopy(lambda o, n: accU.at[pl.ds(o, n)],
                 lambda o, n: outU.at[pl.ds(c * NU + o, n)], r0U, rlU)
        if with_cnt:
            @pl.when(jnp.logical_and(s == 0, c == 0))
            def _():
                pltpu.sync_copy(cacR, ocR0)

            @pl.when(jnp.logical_and(s == 0, c == 1))
            def _():
                pltpu.sync_copy(cacR, ocR1)

            @pl.when(jnp.logical_and(s == 1, c == 0))
            def _():
                pltpu.sync_copy(cacU, ocU0)

            @pl.when(jnp.logical_and(s == 1, c == 1))
            def _():
                pltpu.sync_copy(cacU, ocU1)

    return pl.kernel(body, out_type=out_type, mesh=_mesh(), scratch_types=scratch)


def _make_decode(NG, EL, NRR):
    n_full = EL // C
    tail = EL - n_full * C
    k_max = (n_full + NW - 1) // NW
    assert tail % L == 0 and (n_full * C) % 8 == 0

    scratch = [
        pltpu.VMEM((C,), jnp.int32),
        pltpu.VMEM((C,), jnp.int32),
        pltpu.VMEM((C,), jnp.int32),
        pltpu.VMEM((C,), jnp.float32),
        pltpu.SemaphoreType.DMA,
    ]

    def body(g_h, row_h, col_h, out_h, ridx, cidx, fidx, vals, sem):
        c = lax.axis_index("c")
        s = lax.axis_index("s")
        wid = s * NC + c

        def do_chunk(base, n):
            pltpu.sync_copy(row_h.at[pl.ds(base, n)], ridx.at[pl.ds(0, n)])
            pltpu.sync_copy(col_h.at[pl.ds(base, n)], cidx.at[pl.ds(0, n)])
            for j in range(n // L):
                sl = pl.ds(j * L, L)
                fidx[sl] = ridx[sl] * NRR + cidx[sl]
            pltpu.async_copy(g_h.at[fidx], vals, sem).wait()
            pltpu.sync_copy(vals.at[pl.ds(0, n)], out_h.at[pl.ds(base, n)])

        def step(k, carry):
            g = wid + k * NW

            @pl.when(g < n_full)
            def _():
                do_chunk(g * C, C)

            return carry

        lax.fori_loop(0, k_max, step, 0)
        if tail:
            @pl.when(wid == NW - 1)
            def _():
                base = n_full * C
                pltpu.sync_copy(row_h.at[pl.ds(base, tail)], ridx.at[pl.ds(0, tail)])
                pltpu.sync_copy(col_h.at[pl.ds(base, tail)], cidx.at[pl.ds(0, tail)])
                for j in range(tail // L):
                    sl = pl.ds(j * L, L)
                    fidx[sl] = ridx[sl] * NRR + cidx[sl]
                pltpu.async_copy(g_h.at[fidx.at[pl.ds(0, tail)]], vals.at[pl.ds(0, tail)], sem).wait()
                pltpu.sync_copy(vals.at[pl.ds(0, tail)], out_h.at[pl.ds(base, tail)])

    return pl.kernel(
        body,
        out_type=jax.ShapeDtypeStruct((EL,), jnp.float32),
        mesh=_mesh(),
        scratch_types=scratch,
    )


def _side_affine(agg_ref, cnt0_ref, cnt1_ref, x_ref, Wl_ref, bl_ref, Wr_ref):
    N = agg_ref.shape[1]
    agg = agg_ref[0] + agg_ref[1]
    cnt = jnp.maximum(cnt0_ref[...][:N] + cnt1_ref[...][:N], 1.0)
    agg = agg / cnt
    return (lax.dot_general(agg, Wl_ref[...], (((1,), (1,)), ((), ())),
                            precision=_HIGH, preferred_element_type=jnp.float32)
            + bl_ref[...]
            + lax.dot_general(x_ref[...], Wr_ref[...], (((1,), (1,)), ((), ())),
                              precision=_HIGH, preferred_element_type=jnp.float32))


def _tc_side(N, D, relu):
    def body(agg, cnt0, cnt1, x, Wl, bl, Wr, o):
        h = _side_affine(agg, cnt0, cnt1, x, Wl, bl, Wr)
        o[...] = jnp.maximum(h, 0.0) if relu else h

    return pl.pallas_call(
        body, out_shape=jax.ShapeDtypeStruct((N, D), jnp.float32))


def _tc_side_head(N, D):
    def body(agg, cnt0, cnt1, x, Wl, bl, Wr, Wh, bh, o):
        h = _side_affine(agg, cnt0, cnt1, x, Wl, bl, Wr)
        o[...] = (lax.dot_general(h, Wh[...], (((1,), (1,)), ((), ())),
                                  precision=_HIGH, preferred_element_type=jnp.float32)
                  + bh[...])

    return pl.pallas_call(
        body, out_shape=jax.ShapeDtypeStruct((N, D), jnp.float32))


def _tc_gram(NU, NR, D, blk=1000):
    assert NU % blk == 0

    def body(u_ref, p_ref, o_ref):
        o_ref[...] = lax.dot_general(u_ref[...], p_ref[...], (((1,), (1,)), ((), ())),
                                     precision=_HIGH, preferred_element_type=jnp.float32)

    return pl.pallas_call(
        body,
        grid=(NU // blk,),
        in_specs=[pl.BlockSpec((blk, D), lambda i: (i, 0)),
                  pl.BlockSpec((NR, D), lambda i: (0, 0))],
        out_specs=pl.BlockSpec((blk, NR), lambda i: (i, 0)),
        out_shape=jax.ShapeDtypeStruct((NU, NR), jnp.float32),
    )


def kernel(x_user, x_repo, edge_index_fwd, edge_index_rev, edge_label_index,
           Wl1f, bl1f, Wr1f, Wl1r, bl1r, Wr1r,
           Wl2f, bl2f, Wr2f, Wl2r, bl2r, Wr2r,
           Wu, bu, Wp, bp):
    NU, D = x_user.shape
    NR = x_repo.shape[0]
    E = edge_index_fwd.shape[1]
    EL = edge_label_index.shape[1]

    srcf = edge_index_fwd[0].astype(jnp.int32)
    dstf = edge_index_fwd[1].astype(jnp.int32)
    srcr = edge_index_rev[0].astype(jnp.int32)
    dstr = edge_index_rev[1].astype(jnp.int32)
    row = edge_label_index[0].astype(jnp.int32)
    col = edge_label_index[1].astype(jnp.int32)

    NRp = -(-NR // 128) * 128
    NUp = -(-NU // 128) * 128
    zrow = jnp.zeros((max(_part8(NU)[0], _part8(NR)[0]), D), jnp.float32)
    zcR = jnp.zeros((NRp,), jnp.float32)
    zcU = jnp.zeros((NUp,), jnp.float32)
    ones_h = jnp.ones((C,), jnp.float32)

    sage1 = _make_sage_pass(NU, NR, D, E, with_cnt=True)
    aggR1, aggU1, cR0, cR1, cU0, cU1 = sage1(
        x_user, x_repo, srcf, dstf, srcr, dstr, zrow, zcR, zcU, ones_h)
    aggR1 = aggR1.reshape(NC, NR, D)
    aggU1 = aggU1.reshape(NC, NU, D)
    cR0, cR1 = cR0.reshape(NRp, 1), cR1.reshape(NRp, 1)
    cU0, cU1 = cU0.reshape(NUp, 1), cU1.reshape(NUp, 1)

    repo1 = _tc_side(NR, D, relu=True)(
        aggR1, cR0, cR1, x_repo, Wl1f, bl1f.reshape(1, D), Wr1f)
    user1 = _tc_side(NU, D, relu=True)(
        aggU1, cU0, cU1, x_user, Wl1r, bl1r.reshape(1, D), Wr1r)

    sage2 = _make_sage_pass(NU, NR, D, E, with_cnt=False)
    aggR2, aggU2 = sage2(user1, repo1, srcf, dstf, srcr, dstr, zrow)
    aggR2 = aggR2.reshape(NC, NR, D)
    aggU2 = aggU2.reshape(NC, NU, D)

    P = _tc_side_head(NR, D)(
        aggR2, cR0, cR1, repo1, Wl2f, bl2f.reshape(1, D), Wr2f,
        Wp, bp.reshape(1, D))
    U = _tc_side_head(NU, D)(
        aggU2, cU0, cU1, user1, Wl2r, bl2r.reshape(1, D), Wr2r,
        Wu, bu.reshape(1, D))

    G = _tc_gram(NU, NR, D)(U, P)

    out = _make_decode(NU * NR, EL, NR)(G.reshape(-1), row, col)
    return out


# speedup vs baseline: 7.3254x; 1.0370x over previous
import functools

import jax
import jax.numpy as jnp
from jax import lax
from jax.experimental import pallas as pl
from jax.experimental.pallas import tpu as pltpu
from jax.experimental.pallas import tpu_sc as plsc

NC = 2
NS = 16
NW = NC * NS
L = 16
C = 128
NB = 3

_HIGH = lax.Precision.HIGHEST


def _part8(N):
    r0 = (-(-N // NS) + 7) // 8 * 8
    rl = N - (NS - 1) * r0
    assert rl > 0 and rl % 8 == 0
    return r0, rl


def _mesh():
    return plsc.VectorSubcoreMesh(
        core_axis_name="c", subcore_axis_name="s", num_cores=NC, num_subcores=NS)


def _make_sage_pass(NU, NR, D, E, with_cnt):
    assert E % C == 0
    n_chunk = E // C
    k_max = (n_chunk + NW - 1) // NW

    r0R, rlR = _part8(NR)
    r0U, rlU = _part8(NU)
    NRp = -(-NR // 128) * 128
    NUp = -(-NU // 128) * 128

    out_type = [
        jax.ShapeDtypeStruct((NC * NR, D), jnp.float32),
        jax.ShapeDtypeStruct((NC * NU, D), jnp.float32),
    ]
    scratch = [
        pltpu.VMEM_SHARED((NR, D), jnp.float32),
        pltpu.VMEM_SHARED((NU, D), jnp.float32),
        pltpu.VMEM((NB, C), jnp.int32),
        pltpu.VMEM((NB, C), jnp.int32),
        pltpu.VMEM((NB, C, D), jnp.float32),
        pltpu.SemaphoreType.DMA((NB,)),
        pltpu.SemaphoreType.DMA((NB,)),
        pltpu.SemaphoreType.DMA((NB,)),
    ]
    if with_cnt:
        out_type += [
            jax.ShapeDtypeStruct((NRp,), jnp.float32),
            jax.ShapeDtypeStruct((NRp,), jnp.float32),
            jax.ShapeDtypeStruct((NUp,), jnp.float32),
            jax.ShapeDtypeStruct((NUp,), jnp.float32),
        ]
        scratch += [
            pltpu.VMEM_SHARED((NRp,), jnp.float32),
            pltpu.VMEM_SHARED((NUp,), jnp.float32),
            pltpu.VMEM((C,), jnp.float32),
            pltpu.SemaphoreType.DMA((NB,)),
        ]

    def body(*refs):
        csem = cacR = cacU = ones = None
        if with_cnt:
            (xa, xb, srcf, dstf, srcr, dstr, zrow, zcR, zcU, ones_h,
             outR, outU, ocR0, ocR1, ocU0, ocU1,
             accR, accU, sidx, didx, rows, isem, gsem, ssem,
             cacR, cacU, ones, csem) = refs
        else:
            (xa, xb, srcf, dstf, srcr, dstr, zrow,
             outR, outU,
             accR, accU, sidx, didx, rows, isem, gsem, ssem) = refs
        c = lax.axis_index("c")
        s = lax.axis_index("s")
        wid = s * NC + c

        def _rowcopy(get_src, get_dst, r0, rl):
            @pl.when(s < NS - 1)
            def _():
                pltpu.sync_copy(get_src(s * r0, r0), get_dst(s * r0, r0))

            @pl.when(s == NS - 1)
            def _():
                pltpu.sync_copy(get_src((NS - 1) * r0, rl),
                                get_dst((NS - 1) * r0, rl))

        _rowcopy(lambda o, n: zrow.at[pl.ds(0, n)],
                 lambda o, n: accR.at[pl.ds(o, n)], r0R, rlR)
        _rowcopy(lambda o, n: zrow.at[pl.ds(0, n)],
                 lambda o, n: accU.at[pl.ds(o, n)], r0U, rlU)
        if with_cnt:
            @pl.when(s == 0)
            def _():
                pltpu.sync_copy(zcR, cacR)

            @pl.when(s == 1)
            def _():
                pltpu.sync_copy(zcU, cacU)

            pltpu.sync_copy(ones_h, ones)
        plsc.subcore_barrier()

        def do_dir(src_h, dst_h, x_h, acc, cacc):
            n_outer = (k_max + NB - 1) // NB

            def drain_scatter(b):
                pltpu.make_async_copy(rows.at[b], acc.at[didx.at[b]], ssem.at[b]).wait()
                if cacc is not None:
                    pltpu.make_async_copy(ones, cacc.at[didx.at[b]], csem.at[b]).wait()

            def step(k, carry):
                for b in range(NB):
                    j = k * NB + b
                    g = wid + j * NW
                    gprev = g - NB * NW

                    @pl.when(jnp.logical_and(j >= NB, gprev < n_chunk))
                    def _():
                        drain_scatter(b)

                    @pl.when(g < n_chunk)
                    def _():
                        base = g * C
                        pltpu.async_copy(src_h.at[pl.ds(base, C)], sidx.at[b], isem.at[b])
                        pltpu.async_copy(dst_h.at[pl.ds(base, C)], didx.at[b], isem.at[b])

                for b in range(NB):
                    g = wid + (k * NB + b) * NW

                    @pl.when(g < n_chunk)
                    def _():
                        pltpu.make_async_copy(src_h.at[pl.ds(0, C)], sidx.at[b], isem.at[b]).wait()
                        pltpu.make_async_copy(dst_h.at[pl.ds(0, C)], didx.at[b], isem.at[b]).wait()
                        pltpu.async_copy(x_h.at[sidx.at[b]], rows.at[b], gsem.at[b])

                for b in range(NB):
                    g = wid + (k * NB + b) * NW

                    @pl.when(g < n_chunk)
                    def _():
                        pltpu.make_async_copy(x_h.at[sidx.at[b]], rows.at[b], gsem.at[b]).wait()
                        pltpu.async_copy(rows.at[b], acc.at[didx.at[b]], ssem.at[b], add=True)
                        if cacc is not None:
                            pltpu.async_copy(ones, cacc.at[didx.at[b]], csem.at[b], add=True)

                return carry

            lax.fori_loop(0, n_outer, step, 0)
            for b in range(NB):
                g = wid + ((n_outer - 1) * NB + b) * NW

                @pl.when(g < n_chunk)
                def _():
                    drain_scatter(b)

        do_dir(srcf, dstf, xa, accR, cacR if with_cnt else None)
        do_dir(srcr, dstr, xb, accU, cacU if with_cnt else None)
        plsc.subcore_barrier()

        _rowcopy(lambda o, n: accR.at[pl.ds(o, n)],
                 lambda o, n: outR.at[pl.ds(c * NR + o, n)], r0R, rlR)
        _rowcopy(lambda o, n: accU.at[pl.ds(o, n)],
                 lambda o, n: outU.at[pl.ds(c * NU + o, n)], r0U, rlU)
        if with_cnt:
            @pl.when(jnp.logical_and(s == 0, c == 0))
            def _():
                pltpu.sync_copy(cacR, ocR0)

            @pl.when(jnp.logical_and(s == 0, c == 1))
            def _():
                pltpu.sync_copy(cacR, ocR1)

            @pl.when(jnp.logical_and(s == 1, c == 0))
            def _():
                pltpu.sync_copy(cacU, ocU0)

            @pl.when(jnp.logical_and(s == 1, c == 1))
            def _():
                pltpu.sync_copy(cacU, ocU1)

    return pl.kernel(body, out_type=out_type, mesh=_mesh(), scratch_types=scratch)


def _make_decode(NG, EL, NRR):
    n_full = EL // C
    tail = EL - n_full * C
    k_max = (n_full + NW - 1) // NW
    assert tail % L == 0 and (n_full * C) % 8 == 0

    scratch = [
        pltpu.VMEM((NB, C), jnp.int32),
        pltpu.VMEM((NB, C), jnp.int32),
        pltpu.VMEM((NB, C), jnp.int32),
        pltpu.VMEM((NB, C), jnp.float32),
        pltpu.SemaphoreType.DMA((NB,)),
        pltpu.SemaphoreType.DMA((NB,)),
        pltpu.SemaphoreType.DMA((NB,)),
        pltpu.VMEM((tail,), jnp.int32),
        pltpu.VMEM((tail,), jnp.int32),
        pltpu.VMEM((tail,), jnp.int32),
        pltpu.VMEM((tail,), jnp.float32),
    ]

    def body(g_h, row_h, col_h, out_h, ridx, cidx, fidx, vals,
             isem, gsem, osem, tridx, tcidx, tfidx, tvals):
        c = lax.axis_index("c")
        s = lax.axis_index("s")
        wid = s * NC + c
        n_outer = (k_max + NB - 1) // NB

        def step(k, carry):
            for b in range(NB):
                j = k * NB + b
                g = wid + j * NW
                gprev = g - NB * NW

                @pl.when(jnp.logical_and(j >= NB, gprev < n_full))
                def _():
                    pltpu.make_async_copy(vals.at[b], out_h.at[pl.ds(0, C)],
                                          osem.at[b]).wait()

                @pl.when(g < n_full)
                def _():
                    base = g * C
                    pltpu.async_copy(row_h.at[pl.ds(base, C)], ridx.at[b], isem.at[b])
                    pltpu.async_copy(col_h.at[pl.ds(base, C)], cidx.at[b], isem.at[b])

            for b in range(NB):
                g = wid + (k * NB + b) * NW

                @pl.when(g < n_full)
                def _():
                    pltpu.make_async_copy(row_h.at[pl.ds(0, C)], ridx.at[b], isem.at[b]).wait()
                    pltpu.make_async_copy(col_h.at[pl.ds(0, C)], cidx.at[b], isem.at[b]).wait()
                    for j in range(C // L):
                        sl = pl.ds(j * L, L)
                        fidx[b, sl] = ridx[b, sl] * NRR + cidx[b, sl]
                    pltpu.async_copy(g_h.at[fidx.at[b]], vals.at[b], gsem.at[b])

            for b in range(NB):
                g = wid + (k * NB + b) * NW

                @pl.when(g < n_full)
                def _():
                    pltpu.make_async_copy(g_h.at[fidx.at[b]], vals.at[b], gsem.at[b]).wait()
                    pltpu.async_copy(vals.at[b], out_h.at[pl.ds(g * C, C)], osem.at[b])

            return carry

        lax.fori_loop(0, n_outer, step, 0)
        for b in range(NB):
            g = wid + ((n_outer - 1) * NB + b) * NW

            @pl.when(g < n_full)
            def _():
                pltpu.make_async_copy(vals.at[b], out_h.at[pl.ds(0, C)], osem.at[b]).wait()

        if tail:
            @pl.when(wid == NW - 1)
            def _():
                base = n_full * C
                pltpu.sync_copy(row_h.at[pl.ds(base, tail)], tridx)
                pltpu.sync_copy(col_h.at[pl.ds(base, tail)], tcidx)
                for j in range(tail // L):
                    sl = pl.ds(j * L, L)
                    tfidx[sl] = tridx[sl] * NRR + tcidx[sl]
                pltpu.async_copy(g_h.at[tfidx], tvals, gsem.at[0]).wait()
                pltpu.sync_copy(tvals, out_h.at[pl.ds(base, tail)])

    return pl.kernel(
        body,
        out_type=jax.ShapeDtypeStruct((EL,), jnp.float32),
        mesh=_mesh(),
        scratch_types=scratch,
    )


def _side_affine(agg_ref, cnt0_ref, cnt1_ref, x_ref, Wl_ref, bl_ref, Wr_ref):
    N = agg_ref.shape[1]
    agg = agg_ref[0] + agg_ref[1]
    cnt = jnp.maximum(cnt0_ref[...][:N] + cnt1_ref[...][:N], 1.0)
    agg = agg / cnt
    return (lax.dot_general(agg, Wl_ref[...], (((1,), (1,)), ((), ())),
                            precision=_HIGH, preferred_element_type=jnp.float32)
            + bl_ref[...]
            + lax.dot_general(x_ref[...], Wr_ref[...], (((1,), (1,)), ((), ())),
                              precision=_HIGH, preferred_element_type=jnp.float32))


def _tc_side(N, D, relu):
    def body(agg, cnt0, cnt1, x, Wl, bl, Wr, o):
        h = _side_affine(agg, cnt0, cnt1, x, Wl, bl, Wr)
        o[...] = jnp.maximum(h, 0.0) if relu else h

    return pl.pallas_call(
        body, out_shape=jax.ShapeDtypeStruct((N, D), jnp.float32))


def _tc_side_head(N, D):
    def body(agg, cnt0, cnt1, x, Wl, bl, Wr, Wh, bh, o):
        h = _side_affine(agg, cnt0, cnt1, x, Wl, bl, Wr)
        o[...] = (lax.dot_general(h, Wh[...], (((1,), (1,)), ((), ())),
                                  precision=_HIGH, preferred_element_type=jnp.float32)
                  + bh[...])

    return pl.pallas_call(
        body, out_shape=jax.ShapeDtypeStruct((N, D), jnp.float32))


def _tc_gram(NU, NR, D, blk=1000):
    assert NU % blk == 0

    def body(u_ref, p_ref, o_ref):
        o_ref[...] = lax.dot_general(u_ref[...], p_ref[...], (((1,), (1,)), ((), ())),
                                     precision=_HIGH, preferred_element_type=jnp.float32)

    return pl.pallas_call(
        body,
        grid=(NU // blk,),
        in_specs=[pl.BlockSpec((blk, D), lambda i: (i, 0)),
                  pl.BlockSpec((NR, D), lambda i: (0, 0))],
        out_specs=pl.BlockSpec((blk, NR), lambda i: (i, 0)),
        out_shape=jax.ShapeDtypeStruct((NU, NR), jnp.float32),
    )


def kernel(x_user, x_repo, edge_index_fwd, edge_index_rev, edge_label_index,
           Wl1f, bl1f, Wr1f, Wl1r, bl1r, Wr1r,
           Wl2f, bl2f, Wr2f, Wl2r, bl2r, Wr2r,
           Wu, bu, Wp, bp):
    NU, D = x_user.shape
    NR = x_repo.shape[0]
    E = edge_index_fwd.shape[1]
    EL = edge_label_index.shape[1]

    srcf = edge_index_fwd[0].astype(jnp.int32)
    dstf = edge_index_fwd[1].astype(jnp.int32)
    srcr = edge_index_rev[0].astype(jnp.int32)
    dstr = edge_index_rev[1].astype(jnp.int32)
    row = edge_label_index[0].astype(jnp.int32)
    col = edge_label_index[1].astype(jnp.int32)

    NRp = -(-NR // 128) * 128
    NUp = -(-NU // 128) * 128
    zrow = jnp.zeros((max(_part8(NU)[0], _part8(NR)[0]), D), jnp.float32)
    zcR = jnp.zeros((NRp,), jnp.float32)
    zcU = jnp.zeros((NUp,), jnp.float32)
    ones_h = jnp.ones((C,), jnp.float32)

    sage1 = _make_sage_pass(NU, NR, D, E, with_cnt=True)
    aggR1, aggU1, cR0, cR1, cU0, cU1 = sage1(
        x_user, x_repo, srcf, dstf, srcr, dstr, zrow, zcR, zcU, ones_h)
    aggR1 = aggR1.reshape(NC, NR, D)
    aggU1 = aggU1.reshape(NC, NU, D)
    cR0, cR1 = cR0.reshape(NRp, 1), cR1.reshape(NRp, 1)
    cU0, cU1 = cU0.reshape(NUp, 1), cU1.reshape(NUp, 1)

    repo1 = _tc_side(NR, D, relu=True)(
        aggR1, cR0, cR1, x_repo, Wl1f, bl1f.reshape(1, D), Wr1f)
    user1 = _tc_side(NU, D, relu=True)(
        aggU1, cU0, cU1, x_user, Wl1r, bl1r.reshape(1, D), Wr1r)

    sage2 = _make_sage_pass(NU, NR, D, E, with_cnt=False)
    aggR2, aggU2 = sage2(user1, repo1, srcf, dstf, srcr, dstr, zrow)
    aggR2 = aggR2.reshape(NC, NR, D)
    aggU2 = aggU2.reshape(NC, NU, D)

    P = _tc_side_head(NR, D)(
        aggR2, cR0, cR1, repo1, Wl2f, bl2f.reshape(1, D), Wr2f,
        Wp, bp.reshape(1, D))
    U = _tc_side_head(NU, D)(
        aggU2, cU0, cU1, user1, Wl2r, bl2r.reshape(1, D), Wr2r,
        Wu, bu.reshape(1, D))

    G = _tc_gram(NU, NR, D)(U, P)

    out = _make_decode(NU * NR, EL, NR)(G.reshape(-1), row, col)
    return out


# speedup vs baseline: 7.5021x; 1.0241x over previous
import functools

import jax
import jax.numpy as jnp
from jax import lax
from jax.experimental import pallas as pl
from jax.experimental.pallas import tpu as pltpu
from jax.experimental.pallas import tpu_sc as plsc

NC = 2
NS = 16
NW = NC * NS
L = 16
C = 64
NB = 5

_HIGH = lax.Precision.HIGHEST


def _part8(N):
    r0 = (-(-N // NS) + 7) // 8 * 8
    rl = N - (NS - 1) * r0
    assert rl > 0 and rl % 8 == 0
    return r0, rl


def _mesh():
    return plsc.VectorSubcoreMesh(
        core_axis_name="c", subcore_axis_name="s", num_cores=NC, num_subcores=NS)


def _make_sage_pass(NU, NR, D, E, with_cnt):
    assert E % C == 0
    n_chunk = E // C
    k_max = (n_chunk + NW - 1) // NW

    r0R, rlR = _part8(NR)
    r0U, rlU = _part8(NU)
    NRp = -(-NR // 128) * 128
    NUp = -(-NU // 128) * 128

    out_type = [
        jax.ShapeDtypeStruct((NC * NR, D), jnp.float32),
        jax.ShapeDtypeStruct((NC * NU, D), jnp.float32),
    ]
    scratch = [
        pltpu.VMEM_SHARED((NR, D), jnp.float32),
        pltpu.VMEM_SHARED((NU, D), jnp.float32),
        pltpu.VMEM((NB, C), jnp.int32),
        pltpu.VMEM((NB, C), jnp.int32),
        pltpu.VMEM((NB, C, D), jnp.float32),
        pltpu.SemaphoreType.DMA((NB,)),
        pltpu.SemaphoreType.DMA((NB,)),
        pltpu.SemaphoreType.DMA((NB,)),
    ]
    if with_cnt:
        out_type += [
            jax.ShapeDtypeStruct((NRp,), jnp.float32),
            jax.ShapeDtypeStruct((NRp,), jnp.float32),
            jax.ShapeDtypeStruct((NUp,), jnp.float32),
            jax.ShapeDtypeStruct((NUp,), jnp.float32),
        ]
        scratch += [
            pltpu.VMEM_SHARED((NRp,), jnp.float32),
            pltpu.VMEM_SHARED((NUp,), jnp.float32),
            pltpu.VMEM((C,), jnp.float32),
            pltpu.SemaphoreType.DMA((NB,)),
        ]

    def body(*refs):
        csem = cacR = cacU = ones = None
        if with_cnt:
            (xa, xb, srcf, dstf, srcr, dstr, zrow, zcR, zcU, ones_h,
             outR, outU, ocR0, ocR1, ocU0, ocU1,
             accR, accU, sidx, didx, rows, isem, gsem, ssem,
             cacR, cacU, ones, csem) = refs
        else:
            (xa, xb, srcf, dstf, srcr, dstr, zrow,
             outR, outU,
             accR, accU, sidx, didx, rows, isem, gsem, ssem) = refs
        c = lax.axis_index("c")
        s = lax.axis_index("s")
        wid = s * NC + c

        def _rowcopy(get_src, get_dst, r0, rl):
            @pl.when(s < NS - 1)
            def _():
                pltpu.sync_copy(get_src(s * r0, r0), get_dst(s * r0, r0))

            @pl.when(s == NS - 1)
            def _():
                pltpu.sync_copy(get_src((NS - 1) * r0, rl),
                                get_dst((NS - 1) * r0, rl))

        _rowcopy(lambda o, n: zrow.at[pl.ds(0, n)],
                 lambda o, n: accR.at[pl.ds(o, n)], r0R, rlR)
        _rowcopy(lambda o, n: zrow.at[pl.ds(0, n)],
                 lambda o, n: accU.at[pl.ds(o, n)], r0U, rlU)
        if with_cnt:
            @pl.when(s == 0)
            def _():
                pltpu.sync_copy(zcR, cacR)

            @pl.when(s == 1)
            def _():
                pltpu.sync_copy(zcU, cacU)

            pltpu.sync_copy(ones_h, ones)
        plsc.subcore_barrier()

        def do_dir(src_h, dst_h, x_h, acc, cacc):
            n_outer = (k_max + NB - 1) // NB

            def drain_scatter(b):
                pltpu.make_async_copy(rows.at[b], acc.at[didx.at[b]], ssem.at[b]).wait()
                if cacc is not None:
                    pltpu.make_async_copy(ones, cacc.at[didx.at[b]], csem.at[b]).wait()

            def step(k, carry):
                for b in range(NB):
                    j = k * NB + b
                    g = wid + j * NW
                    gprev = g - NB * NW

                    @pl.when(jnp.logical_and(j >= NB, gprev < n_chunk))
                    def _():
                        drain_scatter(b)

                    @pl.when(g < n_chunk)
                    def _():
                        base = g * C
                        pltpu.async_copy(src_h.at[pl.ds(base, C)], sidx.at[b], isem.at[b])
                        pltpu.async_copy(dst_h.at[pl.ds(base, C)], didx.at[b], isem.at[b])

                for b in range(NB):
                    g = wid + (k * NB + b) * NW

                    @pl.when(g < n_chunk)
                    def _():
                        pltpu.make_async_copy(src_h.at[pl.ds(0, C)], sidx.at[b], isem.at[b]).wait()
                        pltpu.make_async_copy(dst_h.at[pl.ds(0, C)], didx.at[b], isem.at[b]).wait()
                        pltpu.async_copy(x_h.at[sidx.at[b]], rows.at[b], gsem.at[b])

                for b in range(NB):
                    g = wid + (k * NB + b) * NW

                    @pl.when(g < n_chunk)
                    def _():
                        pltpu.make_async_copy(x_h.at[sidx.at[b]], rows.at[b], gsem.at[b]).wait()
                        pltpu.async_copy(rows.at[b], acc.at[didx.at[b]], ssem.at[b], add=True)
                        if cacc is not None:
                            pltpu.async_copy(ones, cacc.at[didx.at[b]], csem.at[b], add=True)

                return carry

            lax.fori_loop(0, n_outer, step, 0)
            for b in range(NB):
                g = wid + ((n_outer - 1) * NB + b) * NW

                @pl.when(g < n_chunk)
                def _():
                    drain_scatter(b)

        do_dir(srcf, dstf, xa, accR, cacR if with_cnt else None)
        do_dir(srcr, dstr, xb, accU, cacU if with_cnt else None)
        plsc.subcore_barrier()

        _rowcopy(lambda o, n: accR.at[pl.ds(o, n)],
                 lambda o, n: outR.at[pl.ds(c * NR + o, n)], r0R, rlR)
        _rowcopy(lambda o, n: accU.at[pl.ds(o, n)],
                 lambda o, n: outU.at[pl.ds(c * NU + o, n)], r0U, rlU)
        if with_cnt:
            @pl.when(jnp.logical_and(s == 0, c == 0))
            def _():
                pltpu.sync_copy(cacR, ocR0)

            @pl.when(jnp.logical_and(s == 0, c == 1))
            def _():
                pltpu.sync_copy(cacR, ocR1)

            @pl.when(jnp.logical_and(s == 1, c == 0))
            def _():
                pltpu.sync_copy(cacU, ocU0)

            @pl.when(jnp.logical_and(s == 1, c == 1))
            def _():
                pltpu.sync_copy(cacU, ocU1)

    return pl.kernel(body, out_type=out_type, mesh=_mesh(), scratch_types=scratch)


DC = 128


def _make_decode(NG, EL, NRR):
    C = DC
    n_full = EL // C
    tail = EL - n_full * C
    k_max = (n_full + NW - 1) // NW
    assert tail % L == 0 and (n_full * C) % 8 == 0

    scratch = [
        pltpu.VMEM((NB, C), jnp.int32),
        pltpu.VMEM((NB, C), jnp.int32),
        pltpu.VMEM((NB, C), jnp.int32),
        pltpu.VMEM((NB, C), jnp.float32),
        pltpu.SemaphoreType.DMA((NB,)),
        pltpu.SemaphoreType.DMA((NB,)),
        pltpu.SemaphoreType.DMA((NB,)),
        pltpu.VMEM((tail,), jnp.int32),
        pltpu.VMEM((tail,), jnp.int32),
        pltpu.VMEM((tail,), jnp.int32),
        pltpu.VMEM((tail,), jnp.float32),
    ]

    def body(g_h, row_h, col_h, out_h, ridx, cidx, fidx, vals,
             isem, gsem, osem, tridx, tcidx, tfidx, tvals):
        c = lax.axis_index("c")
        s = lax.axis_index("s")
        wid = s * NC + c
        n_outer = (k_max + NB - 1) // NB

        def step(k, carry):
            for b in range(NB):
                j = k * NB + b
                g = wid + j * NW
                gprev = g - NB * NW

                @pl.when(jnp.logical_and(j >= NB, gprev < n_full))
                def _():
                    pltpu.make_async_copy(vals.at[b], out_h.at[pl.ds(0, C)],
                                          osem.at[b]).wait()

                @pl.when(g < n_full)
                def _():
                    base = g * C
                    pltpu.async_copy(row_h.at[pl.ds(base, C)], ridx.at[b], isem.at[b])
                    pltpu.async_copy(col_h.at[pl.ds(base, C)], cidx.at[b], isem.at[b])

            for b in range(NB):
                g = wid + (k * NB + b) * NW

                @pl.when(g < n_full)
                def _():
                    pltpu.make_async_copy(row_h.at[pl.ds(0, C)], ridx.at[b], isem.at[b]).wait()
                    pltpu.make_async_copy(col_h.at[pl.ds(0, C)], cidx.at[b], isem.at[b]).wait()
                    for j in range(C // L):
                        sl = pl.ds(j * L, L)
                        fidx[b, sl] = ridx[b, sl] * NRR + cidx[b, sl]
                    pltpu.async_copy(g_h.at[fidx.at[b]], vals.at[b], gsem.at[b])

            for b in range(NB):
                g = wid + (k * NB + b) * NW

                @pl.when(g < n_full)
                def _():
                    pltpu.make_async_copy(g_h.at[fidx.at[b]], vals.at[b], gsem.at[b]).wait()
                    pltpu.async_copy(vals.at[b], out_h.at[pl.ds(g * C, C)], osem.at[b])

            return carry

        lax.fori_loop(0, n_outer, step, 0)
        for b in range(NB):
            g = wid + ((n_outer - 1) * NB + b) * NW

            @pl.when(g < n_full)
            def _():
                pltpu.make_async_copy(vals.at[b], out_h.at[pl.ds(0, C)], osem.at[b]).wait()

        if tail:
            @pl.when(wid == NW - 1)
            def _():
                base = n_full * C
                pltpu.sync_copy(row_h.at[pl.ds(base, tail)], tridx)
                pltpu.sync_copy(col_h.at[pl.ds(base, tail)], tcidx)
                for j in range(tail // L):
                    sl = pl.ds(j * L, L)
                    tfidx[sl] = tridx[sl] * NRR + tcidx[sl]
                pltpu.async_copy(g_h.at[tfidx], tvals, gsem.at[0]).wait()
                pltpu.sync_copy(tvals, out_h.at[pl.ds(base, tail)])

    return pl.kernel(
        body,
        out_type=jax.ShapeDtypeStruct((EL,), jnp.float32),
        mesh=_mesh(),
        scratch_types=scratch,
    )


def _side_affine(agg_ref, cnt0_ref, cnt1_ref, x_ref, Wl_ref, bl_ref, Wr_ref):
    N = agg_ref.shape[1]
    agg = agg_ref[0] + agg_ref[1]
    cnt = jnp.maximum(cnt0_ref[...][:N] + cnt1_ref[...][:N], 1.0)
    agg = agg / cnt
    return (lax.dot_general(agg, Wl_ref[...], (((1,), (1,)), ((), ())),
                            precision=_HIGH, preferred_element_type=jnp.float32)
            + bl_ref[...]
            + lax.dot_general(x_ref[...], Wr_ref[...], (((1,), (1,)), ((), ())),
                              precision=_HIGH, preferred_element_type=jnp.float32))


def _tc_side(N, D, relu):
    def body(agg, cnt0, cnt1, x, Wl, bl, Wr, o):
        h = _side_affine(agg, cnt0, cnt1, x, Wl, bl, Wr)
        o[...] = jnp.maximum(h, 0.0) if relu else h

    return pl.pallas_call(
        body, out_shape=jax.ShapeDtypeStruct((N, D), jnp.float32))


def _tc_side_head(N, D):
    def body(agg, cnt0, cnt1, x, Wl, bl, Wr, Wh, bh, o):
        h = _side_affine(agg, cnt0, cnt1, x, Wl, bl, Wr)
        o[...] = (lax.dot_general(h, Wh[...], (((1,), (1,)), ((), ())),
                                  precision=_HIGH, preferred_element_type=jnp.float32)
                  + bh[...])

    return pl.pallas_call(
        body, out_shape=jax.ShapeDtypeStruct((N, D), jnp.float32))


def _tc_gram(NU, NR, D, blk=1000):
    assert NU % blk == 0

    def body(u_ref, p_ref, o_ref):
        o_ref[...] = lax.dot_general(u_ref[...], p_ref[...], (((1,), (1,)), ((), ())),
                                     precision=_HIGH, preferred_element_type=jnp.float32)

    return pl.pallas_call(
        body,
        grid=(NU // blk,),
        in_specs=[pl.BlockSpec((blk, D), lambda i: (i, 0)),
                  pl.BlockSpec((NR, D), lambda i: (0, 0))],
        out_specs=pl.BlockSpec((blk, NR), lambda i: (i, 0)),
        out_shape=jax.ShapeDtypeStruct((NU, NR), jnp.float32),
    )


def kernel(x_user, x_repo, edge_index_fwd, edge_index_rev, edge_label_index,
           Wl1f, bl1f, Wr1f, Wl1r, bl1r, Wr1r,
           Wl2f, bl2f, Wr2f, Wl2r, bl2r, Wr2r,
           Wu, bu, Wp, bp):
    NU, D = x_user.shape
    NR = x_repo.shape[0]
    E = edge_index_fwd.shape[1]
    EL = edge_label_index.shape[1]

    srcf = edge_index_fwd[0].astype(jnp.int32)
    dstf = edge_index_fwd[1].astype(jnp.int32)
    srcr = edge_index_rev[0].astype(jnp.int32)
    dstr = edge_index_rev[1].astype(jnp.int32)
    row = edge_label_index[0].astype(jnp.int32)
    col = edge_label_index[1].astype(jnp.int32)

    NRp = -(-NR // 128) * 128
    NUp = -(-NU // 128) * 128
    zrow = jnp.zeros((max(_part8(NU)[0], _part8(NR)[0]), D), jnp.float32)
    zcR = jnp.zeros((NRp,), jnp.float32)
    zcU = jnp.zeros((NUp,), jnp.float32)
    ones_h = jnp.ones((C,), jnp.float32)

    sage1 = _make_sage_pass(NU, NR, D, E, with_cnt=True)
    aggR1, aggU1, cR0, cR1, cU0, cU1 = sage1(
        x_user, x_repo, srcf, dstf, srcr, dstr, zrow, zcR, zcU, ones_h)
    aggR1 = aggR1.reshape(NC, NR, D)
    aggU1 = aggU1.reshape(NC, NU, D)
    cR0, cR1 = cR0.reshape(NRp, 1), cR1.reshape(NRp, 1)
    cU0, cU1 = cU0.reshape(NUp, 1), cU1.reshape(NUp, 1)

    repo1 = _tc_side(NR, D, relu=True)(
        aggR1, cR0, cR1, x_repo, Wl1f, bl1f.reshape(1, D), Wr1f)
    user1 = _tc_side(NU, D, relu=True)(
        aggU1, cU0, cU1, x_user, Wl1r, bl1r.reshape(1, D), Wr1r)

    sage2 = _make_sage_pass(NU, NR, D, E, with_cnt=False)
    aggR2, aggU2 = sage2(user1, repo1, srcf, dstf, srcr, dstr, zrow)
    aggR2 = aggR2.reshape(NC, NR, D)
    aggU2 = aggU2.reshape(NC, NU, D)

    P = _tc_side_head(NR, D)(
        aggR2, cR0, cR1, repo1, Wl2f, bl2f.reshape(1, D), Wr2f,
        Wp, bp.reshape(1, D))
    U = _tc_side_head(NU, D)(
        aggU2, cU0, cU1, user1, Wl2r, bl2r.reshape(1, D), Wr2r,
        Wu, bu.reshape(1, D))

    G = _tc_gram(NU, NR, D)(U, P)

    out = _make_decode(NU * NR, EL, NR)(G.reshape(-1), row, col)
    return out
